# Initial kernel scaffold; baseline (speedup 1.0000x reference)
import jax
import jax.numpy as jnp
from jax.experimental import pallas as pl


def kernel(x, expert_weights_v, expert_weights_g, gate_w, gate_b):
    raise NotImplementedError("write your pallas kernel here")


# speedup vs baseline: 2.8351x; 2.8351x over previous
import functools

import jax
import jax.numpy as jnp
from jax import lax
from jax.experimental import pallas as pl
from jax.experimental.pallas import tpu as pltpu
from jax.experimental.pallas import tpu_sc as plsc

B, D, H, E = 2048, 1024, 2048, 8
R = 256
NMB = (B * 2) // R + E
NPAD = NMB * R
HBLK = 1024
NHB = H // HBLK

NC, NS = 2, 16
NW = NC * NS
TPW = B // NW
CH = 16
LN = 16


def _cumsum_lanes(a):
    n = 1
    w = a.shape[1]
    while n < w:
        pad = jnp.zeros((a.shape[0], n), a.dtype)
        a = a + jnp.concatenate([pad, a[:, : w - n]], axis=1)
        n *= 2
    return a


def _cumsum_sublanes(a):
    n = 1
    h = a.shape[0]
    while n < h:
        pad = jnp.zeros((n, a.shape[1]), a.dtype)
        a = a + jnp.concatenate([pad, a[: h - n, :]], axis=0)
        n *= 2
    return a


def _router_body(xt_ref, gw_ref, gb_ref,
                 pos0_ref, pos1_ref, s0_ref, s1_ref, bexp_ref, used_ref):
    gw = gw_ref[...].astype(jnp.bfloat16)
    xt = xt_ref[...].astype(jnp.bfloat16)
    logits = lax.dot_general(
        gw, xt, (((1,), (0,)), ((), ())),
        preferred_element_type=jnp.float32)
    logits = logits + gb_ref[...]
    m = jnp.max(logits, axis=0, keepdims=True)
    p = jnp.exp(logits - m)
    p = p / jnp.sum(p, axis=0, keepdims=True)

    eidx = lax.broadcasted_iota(jnp.int32, (E, B), 0).astype(jnp.float32)
    s1 = jnp.max(p, axis=0, keepdims=True)
    i1 = jnp.min(jnp.where(p == s1, eidx, jnp.float32(E)), axis=0, keepdims=True)
    oh0 = eidx == i1
    p2 = jnp.where(oh0, jnp.float32(-1.0), p)
    s2 = jnp.max(p2, axis=0, keepdims=True)
    i2 = jnp.min(jnp.where(p2 == s2, eidx, jnp.float32(E)), axis=0, keepdims=True)
    oh1 = eidx == i2

    c0 = _cumsum_lanes(oh0.astype(jnp.float32))
    c1 = _cumsum_lanes(oh1.astype(jnp.float32))
    t0 = c0[:, B - 1:B]
    t1 = c1[:, B - 1:B]
    g = t0 + t1
    gp = jnp.floor((g + (R - 1)) / R) * R
    base_incl = _cumsum_sublanes(gp)
    base = base_incl - gp

    pos0 = jnp.sum(jnp.where(oh0, base + c0 - 1.0, 0.0), axis=0, keepdims=True)
    pos1 = jnp.sum(jnp.where(oh1, base + t0 + c1 - 1.0, 0.0), axis=0, keepdims=True)
    pos0_ref[...] = pos0.astype(jnp.int32)
    pos1_ref[...] = pos1.astype(jnp.int32)
    s0_ref[...] = s1
    s1_ref[...] = s2

    total = base_incl[E - 1:E, :]
    usedb = total / R
    used_ref[...] = usedb.astype(jnp.int32)
    bi = lax.broadcasted_iota(jnp.int32, (1, NMB), 1).astype(jnp.float32)
    bi = jnp.minimum(bi, usedb - 1.0)
    bexp = jnp.sum((base <= bi * R).astype(jnp.float32), axis=0, keepdims=True) - 1.0
    bexp_ref[...] = bexp.astype(jnp.int32)


def _router(xt, gw, gb):
    return pl.pallas_call(
        _router_body,
        out_shape=(
            jax.ShapeDtypeStruct((1, B), jnp.int32),
            jax.ShapeDtypeStruct((1, B), jnp.int32),
            jax.ShapeDtypeStruct((1, B), jnp.float32),
            jax.ShapeDtypeStruct((1, B), jnp.float32),
            jax.ShapeDtypeStruct((1, NMB), jnp.int32),
            jax.ShapeDtypeStruct((1, 1), jnp.int32),
        ),
    )(xt, gw, gb)


def _gemm_body(bexp_ref, used_ref, xs_ref, wv_ref, wg_ref, ys_ref):
    m = pl.program_id(1)

    @pl.when(m < used_ref[0])
    def _():
        xb = xs_ref[...]
        wv = wv_ref[0]
        wg = wg_ref[0]
        v = jnp.dot(xb, wv, preferred_element_type=jnp.float32)
        g = jnp.dot(xb, wg, preferred_element_type=jnp.float32)
        ys_ref[...] = v * (1.0 / (1.0 + jnp.exp(-g)))


def _grouped_gemm(bexp, used, xs, wv, wg):
    grid_spec = pltpu.PrefetchScalarGridSpec(
        num_scalar_prefetch=2,
        grid=(NHB, NMB),
        in_specs=[
            pl.BlockSpec((R, D), lambda h, m, be, us: (m, 0)),
            pl.BlockSpec((1, D, HBLK), lambda h, m, be, us: (be[m], 0, h)),
            pl.BlockSpec((1, D, HBLK), lambda h, m, be, us: (be[m], 0, h)),
        ],
        out_specs=pl.BlockSpec((R, HBLK), lambda h, m, be, us: (m, h)),
    )
    return pl.pallas_call(
        _gemm_body,
        grid_spec=grid_spec,
        out_shape=jax.ShapeDtypeStruct((NPAD, H), jnp.float32),
        compiler_params=pltpu.CompilerParams(
            dimension_semantics=("arbitrary", "arbitrary")),
    )(bexp, used, xs, wv, wg)


@functools.lru_cache(maxsize=1)
def _sc_kernels():
    mesh = plsc.VectorSubcoreMesh(core_axis_name="c", subcore_axis_name="s")

    @functools.partial(
        pl.kernel,
        mesh=mesh,
        out_type=jax.ShapeDtypeStruct((NPAD, D), jnp.float32),
        scratch_types=[
            pltpu.VMEM((TPW, D), jnp.float32),
            pltpu.VMEM((TPW,), jnp.int32),
            pltpu.VMEM((TPW,), jnp.int32),
            pltpu.SemaphoreType.DMA,
        ],
    )
    def scatter(x_hbm, pos0_hbm, pos1_hbm, xs_hbm, rows_v, i0_v, i1_v, sem):
        wid = lax.axis_index("s") * NC + lax.axis_index("c")
        base = wid * TPW
        pltpu.sync_copy(x_hbm.at[pl.ds(base, TPW)], rows_v)
        pltpu.sync_copy(pos0_hbm.at[pl.ds(base, TPW)], i0_v)
        pltpu.sync_copy(pos1_hbm.at[pl.ds(base, TPW)], i1_v)
        c0 = pltpu.async_copy(rows_v, xs_hbm.at[i0_v], sem)
        c1 = pltpu.async_copy(rows_v, xs_hbm.at[i1_v], sem)
        c0.wait()
        c1.wait()

    @functools.partial(
        pl.kernel,
        mesh=mesh,
        out_type=jax.ShapeDtypeStruct((B, H), jnp.float32),
        scratch_types=[
            pltpu.VMEM((CH, H), jnp.float32),
            pltpu.VMEM((CH, H), jnp.float32),
            pltpu.VMEM((CH, H), jnp.float32),
            pltpu.VMEM((CH,), jnp.int32),
            pltpu.VMEM((CH,), jnp.int32),
            pltpu.VMEM((CH,), jnp.float32),
            pltpu.VMEM((CH,), jnp.float32),
            pltpu.SemaphoreType.DMA,
        ],
    )
    def combine(ys_hbm, pos0_hbm, pos1_hbm, s0_hbm, s1_hbm, out_hbm,
                ya_v, yb_v, ob_v, ia_v, ib_v, sa_v, sb_v, sem):
        wid = lax.axis_index("s") * NC + lax.axis_index("c")
        base = wid * TPW
        for cc in range(TPW // CH):
            tb = base + cc * CH
            pltpu.sync_copy(pos0_hbm.at[pl.ds(tb, CH)], ia_v)
            pltpu.sync_copy(pos1_hbm.at[pl.ds(tb, CH)], ib_v)
            pltpu.sync_copy(s0_hbm.at[pl.ds(tb, CH)], sa_v)
            pltpu.sync_copy(s1_hbm.at[pl.ds(tb, CH)], sb_v)
            ca = pltpu.async_copy(ys_hbm.at[ia_v], ya_v, sem)
            cb = pltpu.async_copy(ys_hbm.at[ib_v], yb_v, sem)
            ca.wait()
            cb.wait()
            sa = sa_v[...]
            sb = sb_v[...]
            sas = [sa[t] for t in range(CH)]
            sbs = [sb[t] for t in range(CH)]

            def body(hc, _):
                sl = pl.ds(hc * LN, LN)
                for t in range(CH):
                    ob_v[t, sl] = sas[t] * ya_v[t, sl] + sbs[t] * yb_v[t, sl]
                return 0

            lax.fori_loop(0, H // LN, body, 0)
            pltpu.sync_copy(ob_v, out_hbm.at[pl.ds(tb, CH)])

    return scatter, combine


def kernel(x, expert_weights_v, expert_weights_g, gate_w, gate_b):
    xt = x.T
    gb = gate_b.reshape(E, 1)
    pos0, pos1, s0, s1, bexp, used = _router(xt, gate_w, gb)
    pos0 = pos0.reshape(B)
    pos1 = pos1.reshape(B)
    scatter, combine = _sc_kernels()
    xs = scatter(x, pos0, pos1)
    ys = _grouped_gemm(bexp.reshape(NMB), used.reshape(1),
                       xs, expert_weights_v, expert_weights_g)
    return combine(ys, pos0, pos1, s0.reshape(B), s1.reshape(B))
